# Initial kernel scaffold; baseline (speedup 1.0000x reference)
import jax
import jax.numpy as jnp
from jax.experimental import pallas as pl


def kernel(x, N, pos_edge_index, neg_edge_index, head, W_enc, b_enc):
    raise NotImplementedError("write your pallas kernel here")


# speedup vs baseline: 4.2233x; 4.2233x over previous
import functools

import numpy as np
import jax
import jax.numpy as jnp
from jax import lax
from jax.experimental import pallas as pl
from jax.experimental.pallas import tpu as pltpu
from jax.experimental.pallas import tpu_sc as plsc

_AUG = 0.1
_D = 128
_NS = 16
_NC = 2
_CHUNK = 128
_TBL = 10112
_STRIPE = _TBL // _NS
_IDXBLK = 16

_CORE_SETS = ((0, 3), (1, 2))


@functools.lru_cache(maxsize=None)
def _edge_plan(Mpos, Mneg, N):
    with jax.ensure_compile_time_eval(), \
         jax.default_device(jax.devices("cpu")[0]):
        key = jax.random.key(1)
        ks = jax.random.split(key, 8)

        def perm(k, M):
            return np.asarray(jax.random.permutation(k, M))

        pos_tM = int(Mpos * _AUG)
        neg_tM = pos_tM
        kpos = int(Mpos * (1 - _AUG))
        kneg = int(Mneg * (1 - _AUG))

        permA = perm(ks[0], Mpos)
        permB = perm(ks[1], Mneg)
        sample = np.asarray(
            jax.random.randint(ks[2], (2, pos_tM + neg_tM), 0, N),
            dtype=np.int32)
        permC = perm(ks[3], Mpos)
        permD = perm(ks[4], Mneg)

    off_neg = Mpos
    off_smp = Mpos + Mneg
    dummy = Mpos + Mneg + sample.shape[1]

    sets = [
        np.concatenate([permA[:kpos], off_smp + np.arange(pos_tM)]),
        np.concatenate([permC[:kpos], off_neg + permD[kneg:]]),
        np.concatenate([off_neg + permB[:kneg],
                        off_smp + np.arange(pos_tM, pos_tM + neg_tM)]),
        np.concatenate([off_neg + permD[:kneg], permC[kpos:]]),
    ]
    chunks = []
    cpads = []
    padded = []
    for s in sets:
        C = -(-len(s) // (_NS * _CHUNK))
        Cp = -(-C // _IDXBLK) * _IDXBLK
        chunks.append(C)
        cpads.append(Cp)
        P = _NS * C * _CHUNK
        s = np.concatenate([s, np.full((P - len(s),), dummy, np.int64)])
        s = s.reshape(_NS, C, _CHUNK)
        s = np.concatenate(
            [s, np.full((_NS, Cp - C, _CHUNK), dummy, np.int64)], axis=1)
        padded.append(s.reshape(-1))
    all_idx = np.concatenate(padded).astype(np.int32)
    offs = np.cumsum([0] + [_NS * Cp * _CHUNK for Cp in cpads])[:-1]
    return (jnp.asarray(sample), jnp.asarray(all_idx),
            tuple(chunks), tuple(cpads), tuple(int(o) for o in offs))


_edge_plan(256000, 64000, 10000)


def _sc_aggregate(x, slots, chunks, N):
    mesh = plsc.VectorSubcoreMesh(core_axis_name="c", subcore_axis_name="s",
                                  num_cores=_NC, num_subcores=_NS)

    @functools.partial(
        pl.kernel,
        out_type=[
            jax.ShapeDtypeStruct((4, _TBL, _D), jnp.float32),
            jax.ShapeDtypeStruct((4, _TBL, _D), jnp.float32),
        ],
        mesh=mesh,
        scratch_types=[
            pltpu.VMEM_SHARED((_TBL, _D), jnp.float32),
            pltpu.VMEM((_IDXBLK, _CHUNK), jnp.int32),
            pltpu.VMEM((_IDXBLK, _CHUNK), jnp.int32),
            pltpu.VMEM((_CHUNK, _D), jnp.float32),
            pltpu.SemaphoreType.DMA,
        ],
    )
    def k(x_hbm, slot0, slot1, agg_out, deg_out,
          agg_t, src_v, dst_v, rows_v, sem):
        cid = lax.axis_index("c")
        sid = lax.axis_index("s")

        z16 = jnp.zeros((16,), jnp.float32)
        o16 = jnp.ones((16,), jnp.float32)

        def fill_zero(i, _):
            for cc in range(_D // 16):
                rows_v[i, pl.ds(cc * 16, 16)] = z16
            return 0

        def fill_one(i, _):
            for cc in range(_D // 16):
                rows_v[i, pl.ds(cc * 16, 16)] = o16
            return 0

        def zero_table():
            lax.fori_loop(0, _CHUNK, fill_zero, 0)
            base = sid * _STRIPE
            n_full = _STRIPE // _CHUNK
            rem = _STRIPE - n_full * _CHUNK
            for r in range(n_full):
                pltpu.sync_copy(rows_v, agg_t.at[pl.ds(base + r * _CHUNK, _CHUNK)])
            if rem:
                pltpu.sync_copy(rows_v.at[pl.ds(0, rem)],
                                agg_t.at[pl.ds(base + n_full * _CHUNK, rem)])

        def feat_body(j, _):
            pltpu.async_copy(x_hbm.at[src_v.at[j]], rows_v, sem).wait()
            pltpu.sync_copy(rows_v, agg_t.at[dst_v.at[j]], add=True)
            return 0

        def deg_body(j, _):
            pltpu.sync_copy(rows_v, agg_t.at[dst_v.at[j]], add=True)
            return 0

        def process(slot_arr, C, body, with_src):
            nblk = lax.div(C + (_IDXBLK - 1), _IDXBLK)
            src_in = slot_arr.at[cid, sid, 0]
            dst_in = slot_arr.at[cid, sid, 1]

            def blk_body(b, _):
                if with_src:
                    pltpu.sync_copy(src_in.at[pl.ds(b * _IDXBLK, _IDXBLK)], src_v)
                pltpu.sync_copy(dst_in.at[pl.ds(b * _IDXBLK, _IDXBLK)], dst_v)
                nb = jnp.minimum(_IDXBLK, C - b * _IDXBLK)
                lax.fori_loop(0, nb, body, 0)
                return 0
            lax.fori_loop(0, nblk, blk_body, 0)

        def dump(out_ref, set_id):
            base = sid * _STRIPE
            pltpu.sync_copy(agg_t.at[pl.ds(base, _STRIPE)],
                            out_ref.at[set_id].at[pl.ds(base, _STRIPE)])

        c0 = jnp.int32(chunks[0])
        c1 = jnp.int32(chunks[1])
        c23 = jnp.int32(chunks[2])
        for arr, C, set_id in ((slot0, jnp.where(cid == 0, c0, c1), cid),
                               (slot1, c23, 3 - cid)):
            zero_table()
            plsc.subcore_barrier()
            process(arr, C, feat_body, True)
            plsc.subcore_barrier()
            dump(agg_out, set_id)
            zero_table()
            lax.fori_loop(0, _CHUNK, fill_one, 0)
            plsc.subcore_barrier()
            process(arr, C, deg_body, False)
            plsc.subcore_barrier()
            dump(deg_out, set_id)

    return k(x, slots[0], slots[1])


def _finalize(aggx, deg, W, b, N):
    R = 1000
    grid = (N // R,)
    b2 = b.reshape(1, _D)

    def body(agg_ref, deg_ref, w_ref, b_ref, m4_ref, xc_ref, o0, o1, o2, o3):
        w = w_ref[...]
        bb = b_ref[...]
        outs = (o0, o1, o2, o3)
        for kk in range(4):
            a = agg_ref[kk]
            d = deg_ref[kk][:, 0:1]
            z = jnp.dot(a, w, preferred_element_type=jnp.float32) + d * bb
            r = jnp.maximum(z, 0.0)
            m4_ref[kk] = r
            on = r / jnp.maximum(d, 1.0)
            outs[kk][...] = on
            xc_ref[:, kk * _D:(kk + 1) * _D] = on

    out = pl.pallas_call(
        body,
        grid=grid,
        in_specs=[
            pl.BlockSpec((4, R, _D), lambda i: (0, i, 0)),
            pl.BlockSpec((4, R, _D), lambda i: (0, i, 0)),
            pl.BlockSpec((_D, _D), lambda i: (0, 0)),
            pl.BlockSpec((1, _D), lambda i: (0, 0)),
        ],
        out_specs=[
            pl.BlockSpec((4, R, _D), lambda i: (0, i, 0)),
            pl.BlockSpec((R, 4 * _D), lambda i: (i, 0)),
            pl.BlockSpec((R, _D), lambda i: (i, 0)),
            pl.BlockSpec((R, _D), lambda i: (i, 0)),
            pl.BlockSpec((R, _D), lambda i: (i, 0)),
            pl.BlockSpec((R, _D), lambda i: (i, 0)),
        ],
        out_shape=[
            jax.ShapeDtypeStruct((4, N, _D), jnp.float32),
            jax.ShapeDtypeStruct((N, 4 * _D), jnp.float32),
            jax.ShapeDtypeStruct((N, _D), jnp.float32),
            jax.ShapeDtypeStruct((N, _D), jnp.float32),
            jax.ShapeDtypeStruct((N, _D), jnp.float32),
            jax.ShapeDtypeStruct((N, _D), jnp.float32),
        ],
    )(aggx, deg, W, b2)
    return out


def kernel(x, N, pos_edge_index, neg_edge_index, head, W_enc, b_enc):
    del N, head
    N_static = x.shape[0]
    Mpos = pos_edge_index.shape[1]
    Mneg = neg_edge_index.shape[1]

    sample, all_idx, chunks, cpads, offs = _edge_plan(Mpos, Mneg, N_static)

    dummy_col = jnp.array([[0], [N_static]], jnp.int32)
    pool = jnp.concatenate(
        [pos_edge_index.astype(jnp.int32), neg_edge_index.astype(jnp.int32),
         sample, dummy_col], axis=1)
    src_all = jnp.take(pool[0], all_idx)
    dst_all = jnp.take(pool[1], all_idx)

    idxs = []
    for Cp, off in zip(cpads, offs):
        P = _NS * Cp * _CHUNK
        s3 = lax.slice(src_all, (off,), (off + P,)).reshape(_NS, Cp, _CHUNK)
        d3 = lax.slice(dst_all, (off,), (off + P,)).reshape(_NS, Cp, _CHUNK)
        idxs.append(jnp.stack([s3, d3], axis=1))

    slots = [jnp.stack([idxs[0], idxs[1]]), jnp.stack([idxs[3], idxs[2]])]
    aggx, deg = _sc_aggregate(x, slots, chunks, N_static)
    m4, xc, o0, o1, o2, o3 = _finalize(aggx, deg, W_enc, b_enc, N_static)
    m = m4.reshape(4 * N_static, _D)
    return (m, xc, o0, o1, o2, o3)


# speedup vs baseline: 4.3793x; 1.0369x over previous
import functools

import numpy as np
import jax
import jax.numpy as jnp
from jax import lax
from jax.experimental import pallas as pl
from jax.experimental.pallas import tpu as pltpu
from jax.experimental.pallas import tpu_sc as plsc

_AUG = 0.1
_D = 128
_NS = 16
_NC = 2
_CHUNK = 128
_TBL = 10112
_STRIPE = _TBL // _NS
_IDXBLK = 16

_CORE_SETS = ((0, 3), (1, 2))


@functools.lru_cache(maxsize=None)
def _edge_plan(Mpos, Mneg, N):
    with jax.ensure_compile_time_eval(), \
         jax.default_device(jax.devices("cpu")[0]):
        key = jax.random.key(1)
        ks = jax.random.split(key, 8)

        def perm(k, M):
            return np.asarray(jax.random.permutation(k, M))

        pos_tM = int(Mpos * _AUG)
        neg_tM = pos_tM
        kpos = int(Mpos * (1 - _AUG))
        kneg = int(Mneg * (1 - _AUG))

        permA = perm(ks[0], Mpos)
        permB = perm(ks[1], Mneg)
        sample = np.asarray(
            jax.random.randint(ks[2], (2, pos_tM + neg_tM), 0, N),
            dtype=np.int32)
        permC = perm(ks[3], Mpos)
        permD = perm(ks[4], Mneg)

    off_neg = Mpos
    off_smp = Mpos + Mneg
    dummy = Mpos + Mneg + sample.shape[1]

    sets = [
        np.concatenate([permA[:kpos], off_smp + np.arange(pos_tM)]),
        np.concatenate([permC[:kpos], off_neg + permD[kneg:]]),
        np.concatenate([off_neg + permB[:kneg],
                        off_smp + np.arange(pos_tM, pos_tM + neg_tM)]),
        np.concatenate([off_neg + permD[:kneg], permC[kpos:]]),
    ]
    chunks = []
    cpads = []
    padded = []
    for s in sets:
        C = -(-len(s) // (_NS * _CHUNK))
        Cp = -(-C // _IDXBLK) * _IDXBLK
        chunks.append(C)
        cpads.append(Cp)
        P = _NS * C * _CHUNK
        s = np.concatenate([s, np.full((P - len(s),), dummy, np.int64)])
        s = s.reshape(_NS, C, _CHUNK)
        s = np.concatenate(
            [s, np.full((_NS, Cp - C, _CHUNK), dummy, np.int64)], axis=1)
        padded.append(s.reshape(-1))
    all_idx = np.concatenate(padded).astype(np.int32)
    offs = np.cumsum([0] + [_NS * Cp * _CHUNK for Cp in cpads])[:-1]
    return (jnp.asarray(sample), jnp.asarray(all_idx),
            tuple(chunks), tuple(cpads), tuple(int(o) for o in offs))


_edge_plan(256000, 64000, 10000)


def _sc_aggregate(x, slots, chunks, N):
    mesh = plsc.VectorSubcoreMesh(core_axis_name="c", subcore_axis_name="s",
                                  num_cores=_NC, num_subcores=_NS)

    @functools.partial(
        pl.kernel,
        out_type=[
            jax.ShapeDtypeStruct((4, _TBL, _D), jnp.float32),
            jax.ShapeDtypeStruct((4, _TBL, _D), jnp.float32),
        ],
        mesh=mesh,
        scratch_types=[
            pltpu.VMEM_SHARED((_TBL, _D), jnp.float32),
            pltpu.VMEM((_IDXBLK, _CHUNK), jnp.int32),
            pltpu.VMEM((_IDXBLK, _CHUNK), jnp.int32),
            pltpu.VMEM((_CHUNK, _D), jnp.float32),
            pltpu.VMEM((_CHUNK, _D), jnp.float32),
            pltpu.SemaphoreType.DMA,
            pltpu.SemaphoreType.DMA,
        ],
    )
    def k(x_hbm, slot0, slot1, agg_out, deg_out,
          agg_t, src_v, dst_v, rows_v, rows_v2, sem, sem2):
        cid = lax.axis_index("c")
        sid = lax.axis_index("s")

        z16 = jnp.zeros((16,), jnp.float32)
        o16 = jnp.ones((16,), jnp.float32)

        def fill_zero(i, _):
            for cc in range(_D // 16):
                rows_v[i, pl.ds(cc * 16, 16)] = z16
            return 0

        def fill_one(i, _):
            for cc in range(_D // 16):
                rows_v[i, pl.ds(cc * 16, 16)] = o16
            return 0

        def zero_table():
            lax.fori_loop(0, _CHUNK, fill_zero, 0)
            base = sid * _STRIPE
            n_full = _STRIPE // _CHUNK
            rem = _STRIPE - n_full * _CHUNK
            for r in range(n_full):
                pltpu.sync_copy(rows_v, agg_t.at[pl.ds(base + r * _CHUNK, _CHUNK)])
            if rem:
                pltpu.sync_copy(rows_v.at[pl.ds(0, rem)],
                                agg_t.at[pl.ds(base + n_full * _CHUNK, rem)])

        def feat_body(j, _):
            pltpu.async_copy(x_hbm.at[src_v.at[j]], rows_v, sem).wait()
            pltpu.sync_copy(rows_v, agg_t.at[dst_v.at[j]], add=True)
            return 0

        def feat_pair(p, _):
            j = 2 * p
            g0 = pltpu.make_async_copy(x_hbm.at[src_v.at[j]], rows_v, sem)
            g1 = pltpu.make_async_copy(x_hbm.at[src_v.at[j + 1]], rows_v2, sem2)
            g0.start()
            g1.start()
            g0.wait()
            pltpu.sync_copy(rows_v, agg_t.at[dst_v.at[j]], add=True)
            g1.wait()
            pltpu.sync_copy(rows_v2, agg_t.at[dst_v.at[j + 1]], add=True)
            return 0

        def deg_body(j, _):
            pltpu.sync_copy(rows_v, agg_t.at[dst_v.at[j]], add=True)
            return 0

        def process(slot_arr, C, body, with_src):
            nblk = lax.div(C + (_IDXBLK - 1), _IDXBLK)
            src_in = slot_arr.at[cid, sid, 0]
            dst_in = slot_arr.at[cid, sid, 1]

            def blk_body(b, _):
                if with_src:
                    pltpu.sync_copy(src_in.at[pl.ds(b * _IDXBLK, _IDXBLK)], src_v)
                pltpu.sync_copy(dst_in.at[pl.ds(b * _IDXBLK, _IDXBLK)], dst_v)
                nb = jnp.minimum(_IDXBLK, C - b * _IDXBLK)
                if with_src:
                    lax.fori_loop(0, nb // 2, feat_pair, 0)
                    lax.fori_loop(2 * (nb // 2), nb, body, 0)
                else:
                    lax.fori_loop(0, nb, body, 0)
                return 0
            lax.fori_loop(0, nblk, blk_body, 0)

        def dump(out_ref, set_id):
            base = sid * _STRIPE
            pltpu.sync_copy(agg_t.at[pl.ds(base, _STRIPE)],
                            out_ref.at[set_id].at[pl.ds(base, _STRIPE)])

        c0 = jnp.int32(chunks[0])
        c1 = jnp.int32(chunks[1])
        c23 = jnp.int32(chunks[2])
        for arr, C, set_id in ((slot0, jnp.where(cid == 0, c0, c1), cid),
                               (slot1, c23, 3 - cid)):
            zero_table()
            plsc.subcore_barrier()
            process(arr, C, feat_body, True)
            plsc.subcore_barrier()
            dump(agg_out, set_id)
            zero_table()
            lax.fori_loop(0, _CHUNK, fill_one, 0)
            plsc.subcore_barrier()
            process(arr, C, deg_body, False)
            plsc.subcore_barrier()
            dump(deg_out, set_id)

    return k(x, slots[0], slots[1])


def _finalize(aggx, deg, W, b, N):
    R = 1000
    grid = (N // R,)
    b2 = b.reshape(1, _D)

    def body(agg_ref, deg_ref, w_ref, b_ref, m4_ref, xc_ref, o0, o1, o2, o3):
        w = w_ref[...]
        bb = b_ref[...]
        outs = (o0, o1, o2, o3)
        for kk in range(4):
            a = agg_ref[kk]
            d = deg_ref[kk][:, 0:1]
            z = jnp.dot(a, w, preferred_element_type=jnp.float32) + d * bb
            r = jnp.maximum(z, 0.0)
            m4_ref[kk] = r
            on = r / jnp.maximum(d, 1.0)
            outs[kk][...] = on
            xc_ref[:, kk * _D:(kk + 1) * _D] = on

    out = pl.pallas_call(
        body,
        grid=grid,
        in_specs=[
            pl.BlockSpec((4, R, _D), lambda i: (0, i, 0)),
            pl.BlockSpec((4, R, _D), lambda i: (0, i, 0)),
            pl.BlockSpec((_D, _D), lambda i: (0, 0)),
            pl.BlockSpec((1, _D), lambda i: (0, 0)),
        ],
        out_specs=[
            pl.BlockSpec((4, R, _D), lambda i: (0, i, 0)),
            pl.BlockSpec((R, 4 * _D), lambda i: (i, 0)),
            pl.BlockSpec((R, _D), lambda i: (i, 0)),
            pl.BlockSpec((R, _D), lambda i: (i, 0)),
            pl.BlockSpec((R, _D), lambda i: (i, 0)),
            pl.BlockSpec((R, _D), lambda i: (i, 0)),
        ],
        out_shape=[
            jax.ShapeDtypeStruct((4, N, _D), jnp.float32),
            jax.ShapeDtypeStruct((N, 4 * _D), jnp.float32),
            jax.ShapeDtypeStruct((N, _D), jnp.float32),
            jax.ShapeDtypeStruct((N, _D), jnp.float32),
            jax.ShapeDtypeStruct((N, _D), jnp.float32),
            jax.ShapeDtypeStruct((N, _D), jnp.float32),
        ],
    )(aggx, deg, W, b2)
    return out


def kernel(x, N, pos_edge_index, neg_edge_index, head, W_enc, b_enc):
    del N, head
    N_static = x.shape[0]
    Mpos = pos_edge_index.shape[1]
    Mneg = neg_edge_index.shape[1]

    sample, all_idx, chunks, cpads, offs = _edge_plan(Mpos, Mneg, N_static)

    dummy_col = jnp.array([[0], [N_static]], jnp.int32)
    pool = jnp.concatenate(
        [pos_edge_index.astype(jnp.int32), neg_edge_index.astype(jnp.int32),
         sample, dummy_col], axis=1)
    src_all = jnp.take(pool[0], all_idx)
    dst_all = jnp.take(pool[1], all_idx)

    idxs = []
    for Cp, off in zip(cpads, offs):
        P = _NS * Cp * _CHUNK
        s3 = lax.slice(src_all, (off,), (off + P,)).reshape(_NS, Cp, _CHUNK)
        d3 = lax.slice(dst_all, (off,), (off + P,)).reshape(_NS, Cp, _CHUNK)
        idxs.append(jnp.stack([s3, d3], axis=1))

    slots = [jnp.stack([idxs[0], idxs[1]]), jnp.stack([idxs[3], idxs[2]])]
    aggx, deg = _sc_aggregate(x, slots, chunks, N_static)
    m4, xc, o0, o1, o2, o3 = _finalize(aggx, deg, W_enc, b_enc, N_static)
    m = m4.reshape(4 * N_static, _D)
    return (m, xc, o0, o1, o2, o3)
